# baseline (speedup 1.0000x reference)
import functools

import jax
import jax.numpy as jnp
from jax import lax
from jax.experimental import pallas as pl
from jax.experimental.pallas import tpu as pltpu
from jax.experimental.pallas import tpu_sc as plsc

NC = 2
NS = 16
L = 16
NPAD = 10240
RPT = NPAD // NS
KDEG = 2000
KAGG = 2000
NBUF = 2


def _newton_rsqrt(d):
  i = lax.bitcast_convert_type(d, jnp.int32)
  i = 0x5F3759DF - lax.shift_right_arithmetic(i, 1)
  y = lax.bitcast_convert_type(i, jnp.float32)
  for _ in range(4):
    y = y * (1.5 - 0.5 * d * y * y)
  return y


_SPLAT_DNUMS = lax.GatherDimensionNumbers(
    offset_dims=(), collapsed_slice_dims=(0,), start_index_map=(0,))


def _splat(vec, jj):
  idx = jnp.full((L, 1), jj, jnp.int32)
  return lax.gather(vec, idx, _SPLAT_DNUMS, (1,),
                    mode=lax.GatherScatterMode.PROMISE_IN_BOUNDS)


@functools.partial(jax.jit, static_argnames=("e_per_tile",))
def _sc_degree(edge_index, *, e_per_tile):
  mesh = plsc.VectorSubcoreMesh(core_axis_name="c", subcore_axis_name="s")
  nch = e_per_tile // KDEG

  @functools.partial(
      pl.kernel,
      mesh=mesh,
      out_type=jax.ShapeDtypeStruct((NC, NPAD), jnp.float32),
      compiler_params=pltpu.CompilerParams(use_tc_tiling_on_sc=False),
      scratch_types=[
          pltpu.VMEM((nch, KDEG), jnp.int32),
          pltpu.VMEM((KDEG,), jnp.float32),
          pltpu.VMEM_SHARED((NPAD,), jnp.float32),
          pltpu.SemaphoreType.DMA,
          pltpu.SemaphoreType.DMA,
          pltpu.SemaphoreType.DMA,
      ],
  )
  def body(ei_hbm, degp_hbm, idx_v, ones_v, acc_sh, isem, s0, s1):
    c = lax.axis_index("c")
    s = lax.axis_index("s")
    sl = pl.ds(s * RPT, RPT)
    base = (c * NS + s) * e_per_tile
    idma = [pltpu.async_copy(ei_hbm.at[1, pl.ds(base + i * KDEG, KDEG)],
                             idx_v.at[i], isem) for i in range(nch)]

    def fill_zero(i, carry):
      ones_v[pl.ds(i * L, L)] = jnp.zeros((L,), jnp.float32)
      return carry

    lax.fori_loop(0, RPT // L, fill_zero, 0)
    pltpu.sync_copy(ones_v.at[pl.ds(0, RPT)], acc_sh.at[sl])

    def fill_one(i, carry):
      ones_v[pl.ds(i * L, L)] = jnp.ones((L,), jnp.float32)
      return carry

    lax.fori_loop(0, KDEG // L, fill_one, 0)
    for d in idma:
      d.wait()
    plsc.subcore_barrier()
    ssems = [s0, s1]
    sdma = []
    for i in range(nch):
      if i >= 2:
        sdma[i - 2].wait()
      sdma.append(pltpu.async_copy(ones_v, acc_sh.at[idx_v.at[i]],
                                   ssems[i % 2], add=True))
    for d in sdma[-2:]:
      d.wait()
    plsc.subcore_barrier()
    pltpu.sync_copy(acc_sh.at[sl], degp_hbm.at[c, sl])

  return body(edge_index)


@functools.partial(jax.jit, static_argnames=("e_per_tile",))
def _sc_aggregate(edge_index, h_pack, degp, *, e_per_tile):
  mesh = plsc.VectorSubcoreMesh(core_axis_name="c", subcore_axis_name="s")
  nch = e_per_tile // KAGG

  @functools.partial(
      pl.kernel,
      mesh=mesh,
      out_type=jax.ShapeDtypeStruct((NC, NPAD, L), jnp.float32),
      compiler_params=pltpu.CompilerParams(use_tc_tiling_on_sc=False),
      scratch_types=[
          pltpu.VMEM((RPT,), jnp.float32),
          pltpu.VMEM((RPT,), jnp.float32),
          pltpu.VMEM((RPT, L), jnp.float32),
          pltpu.VMEM((RPT, L), jnp.float32),
          pltpu.VMEM((nch, KAGG), jnp.int32),
          pltpu.VMEM((nch, KAGG), jnp.int32),
          [pltpu.VMEM((KAGG, L), jnp.float32) for _ in range(NBUF)],
          pltpu.VMEM_SHARED((NPAD, L), jnp.float32),
          pltpu.VMEM_SHARED((NPAD, L), jnp.float32),
          pltpu.SemaphoreType.DMA,
          [pltpu.SemaphoreType.DMA for _ in range(NBUF)],
          [pltpu.SemaphoreType.DMA for _ in range(NBUF)],
      ],
  )
  def body(ei_hbm, h_hbm, degp_hbm, aggp_hbm,
           d1_v, dinv_v, h_v, hs_v, sidx_v, didx_v, rows, acc_sh, hs_sh,
           isem, gsems, ssems):
    c = lax.axis_index("c")
    s = lax.axis_index("s")
    sl = pl.ds(s * RPT, RPT)
    base = (c * NS + s) * e_per_tile

    idma = []
    for i in range(nch):
      ch = pl.ds(base + i * KAGG, KAGG)
      idma.append(pltpu.async_copy(ei_hbm.at[0, ch], sidx_v.at[i], isem))
      idma.append(pltpu.async_copy(ei_hbm.at[1, ch], didx_v.at[i], isem))

    pltpu.sync_copy(degp_hbm.at[0, sl], dinv_v)
    pltpu.sync_copy(degp_hbm.at[1, sl], d1_v)
    pltpu.sync_copy(h_hbm.at[sl], h_v)

    def newton(g, carry):
      gsl = pl.ds(g * L, L)
      deg = dinv_v[gsl] + d1_v[gsl] + 1.0
      dinv_v[gsl] = _newton_rsqrt(deg)
      return carry

    lax.fori_loop(0, RPT // L, newton, 0)

    def scale_hs(g, carry):
      dchunk = dinv_v[pl.ds(g * L, L)]
      for jj in range(L):
        j = g * L + jj
        hs_v[j, :] = h_v[j, :] * _splat(dchunk, jj)
      return carry

    lax.fori_loop(0, RPT // L, scale_hs, 0)
    pltpu.sync_copy(hs_v, hs_sh.at[sl])

    @pl.when(c == 0)
    def _():
      pltpu.sync_copy(hs_v, acc_sh.at[sl])

    @pl.when(c != 0)
    def _():
      def fill_zero(i, carry):
        rows[0][i, :] = jnp.zeros((L,), jnp.float32)
        return carry

      lax.fori_loop(0, RPT, fill_zero, 0)
      pltpu.sync_copy(rows[0].at[pl.ds(0, RPT)], acc_sh.at[sl])

    for d in idma:
      d.wait()
    plsc.subcore_barrier()

    sdma = {}
    gdma = {}

    def issue_gather(i):
      b = i % NBUF
      if i - NBUF >= 0:
        sdma[i - NBUF].wait()
      gdma[i] = pltpu.async_copy(hs_sh.at[sidx_v.at[i]], rows[b], gsems[b])

    issue_gather(0)
    if nch > 1:
      issue_gather(1)
    for i in range(nch):
      b = i % NBUF
      gdma[i].wait()
      sdma[i] = pltpu.async_copy(rows[b], acc_sh.at[didx_v.at[i]],
                                 ssems[b], add=True)
      if i + 2 < nch:
        issue_gather(i + 2)
    for i in range(max(0, nch - NBUF), nch):
      sdma[i].wait()
    plsc.subcore_barrier()

    pltpu.sync_copy(acc_sh.at[sl], hs_v)

    def scale_out(g, carry):
      dchunk = dinv_v[pl.ds(g * L, L)]
      for jj in range(L):
        j = g * L + jj
        hs_v[j, :] = hs_v[j, :] * _splat(dchunk, jj)
      return carry

    lax.fori_loop(0, RPT // L, scale_out, 0)
    pltpu.sync_copy(hs_v, aggp_hbm.at[c, sl])

  return body(edge_index, h_pack, degp)


def _tc_matmul(x, w1):
  n = x.shape[0]

  def body(x_ref, w_ref, out_ref):
    h = jnp.dot(x_ref[...], w_ref[...], preferred_element_type=jnp.float32)
    out_ref[pl.ds(0, n), :] = h

  return pl.pallas_call(
      body,
      out_shape=jax.ShapeDtypeStruct((NPAD, w1.shape[1]), jnp.float32),
  )(x, w1)


def _tc_final_packed(aggp2, b1, wl, bl, n8):
  half = NPAD * L // 128
  h_dim = wl.shape[0]
  c_dim = wl.shape[1]

  def body(a_ref, b1_ref, wl_ref, bl_ref, out_ref):
    b1t = jnp.concatenate([b1_ref[...]] * 8, axis=1)
    blt = jnp.concatenate([bl_ref[...]] * 8, axis=1)
    w = wl_ref[...]
    z = jnp.zeros((h_dim, c_dim), jnp.float32)
    wl_big = jnp.concatenate(
        [jnp.concatenate([w if k == j else z for k in range(8)], axis=0)
         for j in range(8)], axis=1)
    v = a_ref[pl.ds(0, half), :] + a_ref[pl.ds(half, half), :] + b1t
    act = jnp.maximum(v, 0.0)
    lg = jnp.dot(act, wl_big, preferred_element_type=jnp.float32)
    sg = 1.0 / (1.0 + jnp.exp(-(lg + blt)))
    out_ref[...] = sg[:out_ref.shape[0]]

  return pl.pallas_call(
      body,
      out_shape=jax.ShapeDtypeStruct((n8, L), jnp.float32),
  )(aggp2, b1, wl, bl)


def kernel(x_muons, edge_index_muons, W1, b1, Wl, bl, generate_jets=0):
  n, d = x_muons.shape
  h_dim = W1.shape[1]
  c_dim = Wl.shape[1]
  e = edge_index_muons.shape[1]
  e_per_tile = e // (NC * NS)

  h = _tc_matmul(x_muons, W1)
  degp = _sc_degree(edge_index_muons, e_per_tile=e_per_tile)
  aggp = _sc_aggregate(edge_index_muons, h, degp,
                       e_per_tile=e_per_tile)

  out_pack = _tc_final_packed(aggp.reshape(NC * NPAD * h_dim // 128, 128),
                              b1.reshape(1, h_dim), Wl,
                              bl.reshape(1, c_dim), n // 8)
  return out_pack.reshape(n, c_dim)


# speedup vs baseline: 1.0530x; 1.0530x over previous
import functools

import jax
import jax.numpy as jnp
from jax import lax
from jax.experimental import pallas as pl
from jax.experimental.pallas import tpu as pltpu
from jax.experimental.pallas import tpu_sc as plsc

NC = 2
NS = 16
L = 16
NPAD = 10240
RPT = NPAD // NS
KDEG = 1000
KAGG = 1000
NBUF = 4


def _newton_rsqrt(d):
  i = lax.bitcast_convert_type(d, jnp.int32)
  i = 0x5F3759DF - lax.shift_right_arithmetic(i, 1)
  y = lax.bitcast_convert_type(i, jnp.float32)
  for _ in range(4):
    y = y * (1.5 - 0.5 * d * y * y)
  return y


_SPLAT_DNUMS = lax.GatherDimensionNumbers(
    offset_dims=(), collapsed_slice_dims=(0,), start_index_map=(0,))


def _splat(vec, jj):
  idx = jnp.full((L, 1), jj, jnp.int32)
  return lax.gather(vec, idx, _SPLAT_DNUMS, (1,),
                    mode=lax.GatherScatterMode.PROMISE_IN_BOUNDS)


@functools.partial(jax.jit, static_argnames=("e_per_tile",))
def _sc_degree(edge_index, *, e_per_tile):
  mesh = plsc.VectorSubcoreMesh(core_axis_name="c", subcore_axis_name="s")
  nch = e_per_tile // KDEG

  @functools.partial(
      pl.kernel,
      mesh=mesh,
      out_type=jax.ShapeDtypeStruct((NC, NPAD), jnp.float32),
      compiler_params=pltpu.CompilerParams(use_tc_tiling_on_sc=False),
      scratch_types=[
          pltpu.VMEM((nch, KDEG), jnp.int32),
          pltpu.VMEM((KDEG,), jnp.float32),
          pltpu.VMEM_SHARED((NPAD,), jnp.float32),
          pltpu.SemaphoreType.DMA,
          pltpu.SemaphoreType.DMA,
          pltpu.SemaphoreType.DMA,
      ],
  )
  def body(ei_hbm, degp_hbm, idx_v, ones_v, acc_sh, isem, s0, s1):
    c = lax.axis_index("c")
    s = lax.axis_index("s")
    sl = pl.ds(s * RPT, RPT)
    base = (c * NS + s) * e_per_tile
    idma = [pltpu.async_copy(ei_hbm.at[1, pl.ds(base + i * KDEG, KDEG)],
                             idx_v.at[i], isem) for i in range(nch)]

    def fill_zero(i, carry):
      ones_v[pl.ds(i * L, L)] = jnp.zeros((L,), jnp.float32)
      return carry

    lax.fori_loop(0, RPT // L, fill_zero, 0)
    pltpu.sync_copy(ones_v.at[pl.ds(0, RPT)], acc_sh.at[sl])

    def fill_one(i, carry):
      ones_v[pl.ds(i * L, L)] = jnp.ones((L,), jnp.float32)
      return carry

    lax.fori_loop(0, KDEG // L, fill_one, 0)
    for d in idma:
      d.wait()
    plsc.subcore_barrier()
    ssems = [s0, s1]
    sdma = []
    for i in range(nch):
      if i >= 2:
        sdma[i - 2].wait()
      sdma.append(pltpu.async_copy(ones_v, acc_sh.at[idx_v.at[i]],
                                   ssems[i % 2], add=True))
    for d in sdma[-2:]:
      d.wait()
    plsc.subcore_barrier()
    pltpu.sync_copy(acc_sh.at[sl], degp_hbm.at[c, sl])

  return body(edge_index)


@functools.partial(jax.jit, static_argnames=("e_per_tile",))
def _sc_aggregate(edge_index, h_pack, degp, *, e_per_tile):
  mesh = plsc.VectorSubcoreMesh(core_axis_name="c", subcore_axis_name="s")
  nch = e_per_tile // KAGG

  @functools.partial(
      pl.kernel,
      mesh=mesh,
      out_type=jax.ShapeDtypeStruct((NC, NPAD, L), jnp.float32),
      compiler_params=pltpu.CompilerParams(use_tc_tiling_on_sc=False),
      scratch_types=[
          pltpu.VMEM((RPT,), jnp.float32),
          pltpu.VMEM((RPT,), jnp.float32),
          pltpu.VMEM((RPT // 8, 128), jnp.float32),
          pltpu.VMEM((RPT, L), jnp.float32),
          pltpu.VMEM((nch, KAGG), jnp.int32),
          pltpu.VMEM((nch, KAGG), jnp.int32),
          [pltpu.VMEM((KAGG, L), jnp.float32) for _ in range(NBUF)],
          pltpu.VMEM_SHARED((NPAD, L), jnp.float32),
          pltpu.VMEM_SHARED((NPAD, L), jnp.float32),
          pltpu.SemaphoreType.DMA,
          [pltpu.SemaphoreType.DMA for _ in range(NBUF)],
          [pltpu.SemaphoreType.DMA for _ in range(NBUF)],
      ],
  )
  def body(ei_hbm, h_hbm, degp_hbm, aggp_hbm,
           d1_v, dinv_v, h_v, hs_v, sidx_v, didx_v, rows, acc_sh, hs_sh,
           isem, gsems, ssems):
    c = lax.axis_index("c")
    s = lax.axis_index("s")
    sl = pl.ds(s * RPT, RPT)
    base = (c * NS + s) * e_per_tile

    idma = []
    for i in range(nch):
      ch = pl.ds(base + i * KAGG, KAGG)
      idma.append(pltpu.async_copy(ei_hbm.at[0, ch], sidx_v.at[i], isem))
      idma.append(pltpu.async_copy(ei_hbm.at[1, ch], didx_v.at[i], isem))

    pltpu.sync_copy(degp_hbm.at[0, sl], dinv_v)
    pltpu.sync_copy(degp_hbm.at[1, sl], d1_v)
    pltpu.sync_copy(h_hbm.at[pl.ds(s * (RPT // 8), RPT // 8)], h_v)

    def newton(g, carry):
      gsl = pl.ds(g * L, L)
      deg = dinv_v[gsl] + d1_v[gsl] + 1.0
      dinv_v[gsl] = _newton_rsqrt(deg)
      return carry

    lax.fori_loop(0, RPT // L, newton, 0)

    def scale_hs(g, carry):
      dchunk = dinv_v[pl.ds(g * L, L)]
      for jj in range(L):
        j = g * L + jj
        row = h_v[2 * g + jj // 8, pl.ds(16 * (jj % 8), L)]
        hs_v[j, :] = row * _splat(dchunk, jj)
      return carry

    lax.fori_loop(0, RPT // L, scale_hs, 0)
    pltpu.sync_copy(hs_v, hs_sh.at[sl])

    @pl.when(c == 0)
    def _():
      pltpu.sync_copy(hs_v, acc_sh.at[sl])

    @pl.when(c != 0)
    def _():
      def fill_zero(i, carry):
        rows[0][i, :] = jnp.zeros((L,), jnp.float32)
        return carry

      lax.fori_loop(0, RPT, fill_zero, 0)
      pltpu.sync_copy(rows[0].at[pl.ds(0, RPT)], acc_sh.at[sl])

    for d in idma:
      d.wait()
    plsc.subcore_barrier()

    sdma = {}
    gdma = {}

    def issue_gather(i):
      b = i % NBUF
      if i - NBUF >= 0:
        sdma[i - NBUF].wait()
      gdma[i] = pltpu.async_copy(hs_sh.at[sidx_v.at[i]], rows[b], gsems[b])

    issue_gather(0)
    if nch > 1:
      issue_gather(1)
    for i in range(nch):
      b = i % NBUF
      gdma[i].wait()
      sdma[i] = pltpu.async_copy(rows[b], acc_sh.at[didx_v.at[i]],
                                 ssems[b], add=True)
      if i + 2 < nch:
        issue_gather(i + 2)
    for i in range(max(0, nch - NBUF), nch):
      sdma[i].wait()
    plsc.subcore_barrier()

    pltpu.sync_copy(acc_sh.at[sl], hs_v)

    def scale_out(g, carry):
      dchunk = dinv_v[pl.ds(g * L, L)]
      for jj in range(L):
        j = g * L + jj
        hs_v[j, :] = hs_v[j, :] * _splat(dchunk, jj)
      return carry

    lax.fori_loop(0, RPT // L, scale_out, 0)
    pltpu.sync_copy(hs_v, aggp_hbm.at[c, sl])

  return body(edge_index, h_pack, degp)


def _tc_matmul_packed(x3, w1):
  n8 = x3.shape[0]

  def body(x_ref, w_ref, out_ref):
    xa = x_ref[...]
    w = w_ref[...]
    for j in range(8):
      xj = xa[:, j, :]
      hj = jnp.dot(xj, w, preferred_element_type=jnp.float32)
      out_ref[pl.ds(0, n8), pl.ds(L * j, L)] = hj

  return pl.pallas_call(
      body,
      out_shape=jax.ShapeDtypeStruct((NPAD // 8, 128), jnp.float32),
  )(x3, w1)


def _tc_final_packed(aggp2, b1, wl, bl, n8):
  half = NPAD * L // 128
  h_dim = wl.shape[0]
  c_dim = wl.shape[1]

  def body(a_ref, b1_ref, wl_ref, bl_ref, out_ref):
    b1t = jnp.concatenate([b1_ref[...]] * 8, axis=1)
    blt = jnp.concatenate([bl_ref[...]] * 8, axis=1)
    w = wl_ref[...]
    z = jnp.zeros((h_dim, c_dim), jnp.float32)
    wl_big = jnp.concatenate(
        [jnp.concatenate([w if k == j else z for k in range(8)], axis=0)
         for j in range(8)], axis=1)
    v = a_ref[pl.ds(0, half), :] + a_ref[pl.ds(half, half), :] + b1t
    act = jnp.maximum(v, 0.0)
    lg = jnp.dot(act, wl_big, preferred_element_type=jnp.float32)
    sg = 1.0 / (1.0 + jnp.exp(-(lg + blt)))
    out_ref[...] = sg[:out_ref.shape[0]]

  return pl.pallas_call(
      body,
      out_shape=jax.ShapeDtypeStruct((n8, L), jnp.float32),
  )(aggp2, b1, wl, bl)


def kernel(x_muons, edge_index_muons, W1, b1, Wl, bl, generate_jets=0):
  n, d = x_muons.shape
  h_dim = W1.shape[1]
  c_dim = Wl.shape[1]
  e = edge_index_muons.shape[1]
  e_per_tile = e // (NC * NS)

  x3 = x_muons.reshape(n // 8, 8, d)

  h_pack = _tc_matmul_packed(x3, W1)
  degp = _sc_degree(edge_index_muons, e_per_tile=e_per_tile)
  aggp = _sc_aggregate(edge_index_muons, h_pack, degp,
                       e_per_tile=e_per_tile)

  out_pack = _tc_final_packed(aggp.reshape(NC * NPAD * h_dim // 128, 128),
                              b1.reshape(1, h_dim), Wl,
                              bl.reshape(1, c_dim), n // 8)
  return out_pack.reshape(n, c_dim)


# speedup vs baseline: 1.0568x; 1.0037x over previous
import functools

import jax
import jax.numpy as jnp
from jax import lax
from jax.experimental import pallas as pl
from jax.experimental.pallas import tpu as pltpu
from jax.experimental.pallas import tpu_sc as plsc

NC = 2
NS = 16
L = 16
NPAD = 10240
RPT = NPAD // NS
KDEG = 2000
KAGG = 1000
NBUF = 4


def _newton_rsqrt(d):
  i = lax.bitcast_convert_type(d, jnp.int32)
  i = 0x5F3759DF - lax.shift_right_arithmetic(i, 1)
  y = lax.bitcast_convert_type(i, jnp.float32)
  for _ in range(4):
    y = y * (1.5 - 0.5 * d * y * y)
  return y


_SPLAT_DNUMS = lax.GatherDimensionNumbers(
    offset_dims=(), collapsed_slice_dims=(0,), start_index_map=(0,))


def _splat(vec, jj):
  idx = jnp.full((L, 1), jj, jnp.int32)
  return lax.gather(vec, idx, _SPLAT_DNUMS, (1,),
                    mode=lax.GatherScatterMode.PROMISE_IN_BOUNDS)


@functools.partial(jax.jit, static_argnames=("e_per_tile",))
def _sc_degree(edge_index, *, e_per_tile):
  mesh = plsc.VectorSubcoreMesh(core_axis_name="c", subcore_axis_name="s")
  nch = e_per_tile // KDEG

  @functools.partial(
      pl.kernel,
      mesh=mesh,
      out_type=jax.ShapeDtypeStruct((NC, NPAD), jnp.float32),
      compiler_params=pltpu.CompilerParams(use_tc_tiling_on_sc=False),
      scratch_types=[
          pltpu.VMEM((nch, KDEG), jnp.int32),
          pltpu.VMEM((KDEG,), jnp.float32),
          pltpu.VMEM_SHARED((NPAD,), jnp.float32),
          pltpu.SemaphoreType.DMA,
          pltpu.SemaphoreType.DMA,
          pltpu.SemaphoreType.DMA,
      ],
  )
  def body(ei_hbm, degp_hbm, idx_v, ones_v, acc_sh, isem, s0, s1):
    c = lax.axis_index("c")
    s = lax.axis_index("s")
    sl = pl.ds(s * RPT, RPT)
    base = (c * NS + s) * e_per_tile
    idma = [pltpu.async_copy(ei_hbm.at[1, pl.ds(base + i * KDEG, KDEG)],
                             idx_v.at[i], isem) for i in range(nch)]

    def fill_zero(i, carry):
      ones_v[pl.ds(i * L, L)] = jnp.zeros((L,), jnp.float32)
      return carry

    lax.fori_loop(0, RPT // L, fill_zero, 0)
    pltpu.sync_copy(ones_v.at[pl.ds(0, RPT)], acc_sh.at[sl])

    def fill_one(i, carry):
      ones_v[pl.ds(i * L, L)] = jnp.ones((L,), jnp.float32)
      return carry

    lax.fori_loop(0, KDEG // L, fill_one, 0)
    for d in idma:
      d.wait()
    plsc.subcore_barrier()
    ssems = [s0, s1]
    sdma = []
    for i in range(nch):
      if i >= 2:
        sdma[i - 2].wait()
      sdma.append(pltpu.async_copy(ones_v, acc_sh.at[idx_v.at[i]],
                                   ssems[i % 2], add=True))
    for d in sdma[-2:]:
      d.wait()
    plsc.subcore_barrier()
    pltpu.sync_copy(acc_sh.at[sl], degp_hbm.at[c, sl])

  return body(edge_index)


@functools.partial(jax.jit, static_argnames=("e_per_tile",))
def _sc_aggregate(edge_index, h_pack, degp, *, e_per_tile):
  mesh = plsc.VectorSubcoreMesh(core_axis_name="c", subcore_axis_name="s")
  nch = e_per_tile // KAGG

  @functools.partial(
      pl.kernel,
      mesh=mesh,
      out_type=jax.ShapeDtypeStruct((NC, NPAD, L), jnp.float32),
      compiler_params=pltpu.CompilerParams(use_tc_tiling_on_sc=False),
      scratch_types=[
          pltpu.VMEM((RPT,), jnp.float32),
          pltpu.VMEM((RPT,), jnp.float32),
          pltpu.VMEM((RPT // 8, 128), jnp.float32),
          pltpu.VMEM((RPT, L), jnp.float32),
          pltpu.VMEM((nch, KAGG), jnp.int32),
          pltpu.VMEM((nch, KAGG), jnp.int32),
          [pltpu.VMEM((KAGG, L), jnp.float32) for _ in range(NBUF)],
          pltpu.VMEM_SHARED((NPAD, L), jnp.float32),
          pltpu.VMEM_SHARED((NPAD, L), jnp.float32),
          pltpu.SemaphoreType.DMA,
          [pltpu.SemaphoreType.DMA for _ in range(NBUF)],
          [pltpu.SemaphoreType.DMA for _ in range(NBUF)],
      ],
  )
  def body(ei_hbm, h_hbm, degp_hbm, aggp_hbm,
           d1_v, dinv_v, h_v, hs_v, sidx_v, didx_v, rows, acc_sh, hs_sh,
           isem, gsems, ssems):
    c = lax.axis_index("c")
    s = lax.axis_index("s")
    sl = pl.ds(s * RPT, RPT)
    base = (c * NS + s) * e_per_tile

    idma = []
    for i in range(nch):
      ch = pl.ds(base + i * KAGG, KAGG)
      idma.append(pltpu.async_copy(ei_hbm.at[0, ch], sidx_v.at[i], isem))
      idma.append(pltpu.async_copy(ei_hbm.at[1, ch], didx_v.at[i], isem))

    pltpu.sync_copy(degp_hbm.at[0, sl], dinv_v)
    pltpu.sync_copy(degp_hbm.at[1, sl], d1_v)
    pltpu.sync_copy(h_hbm.at[pl.ds(s * (RPT // 8), RPT // 8)], h_v)

    def newton(g, carry):
      gsl = pl.ds(g * L, L)
      deg = dinv_v[gsl] + d1_v[gsl] + 1.0
      dinv_v[gsl] = _newton_rsqrt(deg)
      return carry

    lax.fori_loop(0, RPT // L, newton, 0)

    def scale_hs(g, carry):
      dchunk = dinv_v[pl.ds(g * L, L)]
      for jj in range(L):
        j = g * L + jj
        row = h_v[2 * g + jj // 8, pl.ds(16 * (jj % 8), L)]
        hs_v[j, :] = row * _splat(dchunk, jj)
      return carry

    lax.fori_loop(0, RPT // L, scale_hs, 0)
    pltpu.sync_copy(hs_v, hs_sh.at[sl])

    @pl.when(c == 0)
    def _():
      pltpu.sync_copy(hs_v, acc_sh.at[sl])

    @pl.when(c != 0)
    def _():
      def fill_zero(i, carry):
        rows[0][i, :] = jnp.zeros((L,), jnp.float32)
        return carry

      lax.fori_loop(0, RPT, fill_zero, 0)
      pltpu.sync_copy(rows[0].at[pl.ds(0, RPT)], acc_sh.at[sl])

    for d in idma:
      d.wait()
    plsc.subcore_barrier()

    sdma = {}
    gdma = {}

    def issue_gather(i):
      b = i % NBUF
      if i - NBUF >= 0:
        sdma[i - NBUF].wait()
      gdma[i] = pltpu.async_copy(hs_sh.at[sidx_v.at[i]], rows[b], gsems[b])

    issue_gather(0)
    if nch > 1:
      issue_gather(1)
    for i in range(nch):
      b = i % NBUF
      gdma[i].wait()
      sdma[i] = pltpu.async_copy(rows[b], acc_sh.at[didx_v.at[i]],
                                 ssems[b], add=True)
      if i + 2 < nch:
        issue_gather(i + 2)
    for i in range(max(0, nch - NBUF), nch):
      sdma[i].wait()
    plsc.subcore_barrier()

    pltpu.sync_copy(acc_sh.at[sl], hs_v)

    def scale_out(g, carry):
      dchunk = dinv_v[pl.ds(g * L, L)]
      for jj in range(L):
        j = g * L + jj
        hs_v[j, :] = hs_v[j, :] * _splat(dchunk, jj)
      return carry

    lax.fori_loop(0, RPT // L, scale_out, 0)
    pltpu.sync_copy(hs_v, aggp_hbm.at[c, sl])

  return body(edge_index, h_pack, degp)


def _tc_matmul_packed(x3, w1):
  n8 = x3.shape[0]

  def body(x_ref, w_ref, out_ref):
    xa = x_ref[...]
    w = w_ref[...]
    for j in range(8):
      xj = xa[:, j, :]
      hj = jnp.dot(xj, w, preferred_element_type=jnp.float32)
      out_ref[pl.ds(0, n8), pl.ds(L * j, L)] = hj

  return pl.pallas_call(
      body,
      out_shape=jax.ShapeDtypeStruct((NPAD // 8, 128), jnp.float32),
  )(x3, w1)


def _tc_final_packed(aggp2, b1, wl, bl, n8):
  half = NPAD * L // 128
  h_dim = wl.shape[0]
  c_dim = wl.shape[1]

  def body(a_ref, b1_ref, wl_ref, bl_ref, out_ref):
    b1t = jnp.concatenate([b1_ref[...]] * 8, axis=1)
    blt = jnp.concatenate([bl_ref[...]] * 8, axis=1)
    w = wl_ref[...]
    z = jnp.zeros((h_dim, c_dim), jnp.float32)
    wl_big = jnp.concatenate(
        [jnp.concatenate([w if k == j else z for k in range(8)], axis=0)
         for j in range(8)], axis=1)
    v = a_ref[pl.ds(0, half), :] + a_ref[pl.ds(half, half), :] + b1t
    act = jnp.maximum(v, 0.0)
    lg = jnp.dot(act, wl_big, preferred_element_type=jnp.float32)
    sg = 1.0 / (1.0 + jnp.exp(-(lg + blt)))
    out_ref[...] = sg[:out_ref.shape[0]]

  return pl.pallas_call(
      body,
      out_shape=jax.ShapeDtypeStruct((n8, L), jnp.float32),
  )(aggp2, b1, wl, bl)


def kernel(x_muons, edge_index_muons, W1, b1, Wl, bl, generate_jets=0):
  n, d = x_muons.shape
  h_dim = W1.shape[1]
  c_dim = Wl.shape[1]
  e = edge_index_muons.shape[1]
  e_per_tile = e // (NC * NS)

  x3 = x_muons.reshape(n // 8, 8, d)

  h_pack = _tc_matmul_packed(x3, W1)
  degp = _sc_degree(edge_index_muons, e_per_tile=e_per_tile)
  aggp = _sc_aggregate(edge_index_muons, h_pack, degp,
                       e_per_tile=e_per_tile)

  out_pack = _tc_final_packed(aggp.reshape(NC * NPAD * h_dim // 128, 128),
                              b1.reshape(1, h_dim), Wl,
                              bl.reshape(1, c_dim), n // 8)
  return out_pack.reshape(n, c_dim)
